# baseline (speedup 1.0000x reference)
import math

import jax
import jax.numpy as jnp
from jax.experimental import pallas as pl
from jax.experimental.pallas import tpu as pltpu


def _copy_kernel(x_ref, o_ref):
    o_ref[...] = x_ref[...]


def kernel(batch):
    B, F = batch.shape
    blk = math.gcd(B, 2048)
    return pl.pallas_call(
        _copy_kernel,
        grid=(B // blk,),
        in_specs=[pl.BlockSpec((blk, F), lambda i: (i, 0))],
        out_specs=pl.BlockSpec((blk, F), lambda i: (i, 0)),
        out_shape=jax.ShapeDtypeStruct((B, F), batch.dtype),
        compiler_params=pltpu.CompilerParams(
            dimension_semantics=("arbitrary",),
        ),
    )(batch)
